# Initial kernel scaffold; baseline (speedup 1.0000x reference)
import jax
import jax.numpy as jnp
from jax.experimental import pallas as pl


def kernel(times, values, t):
    raise NotImplementedError("write your pallas kernel here")


# speedup vs baseline: 1.3878x; 1.3878x over previous
import functools

import jax
import jax.numpy as jnp
from jax import lax
from jax.experimental import pallas as pl
from jax.experimental.pallas import tpu as pltpu
from jax.experimental.pallas import tpu_sc as plsc

N_BINS_K = 4096
BATCH_K = 16384
L = 16
NW = 32
ROWS_PER_W = BATCH_K // NW
N_TIMES_PAD = 4112
N_CHUNKS = N_TIMES_PAD // L


def _pwc_body(times_hbm, t_hbm, vflat_hbm, out_hbm,
              times_v, t_v, idx_v, col_v, sem):
    wid = lax.axis_index("s") * 2 + lax.axis_index("c")
    base = wid * ROWS_PER_W

    pltpu.sync_copy(times_hbm, times_v)
    pltpu.sync_copy(t_hbm, t_v)
    tval = t_v[...]

    def cnt_body(i, acc):
        chunk = times_v[pl.ds(i * L, L)]
        return acc + jnp.where(chunk <= tval, jnp.int32(1), jnp.int32(0))

    acc = lax.fori_loop(0, N_CHUNKS, cnt_body, jnp.zeros((L,), jnp.int32))

    lanes = lax.iota(jnp.int32, L)
    count = acc
    for k in (8, 4, 2, 1):
        idx_v[pl.ds(0, L)] = count
        count = count + plsc.load_gather(idx_v, [lanes ^ k])

    valid = jnp.logical_and(count >= 1, count <= N_BINS_K)
    idx_c = jnp.clip(count - 1, 0, N_BINS_K - 1)


    def idx_body(c, carry):
        rows = base + c * L + lanes
        idx_v[pl.ds(c * L, L)] = rows * N_BINS_K + idx_c
        return carry

    lax.fori_loop(0, ROWS_PER_W // L, idx_body, jnp.int32(0))

    copies = [
        pltpu.async_copy(vflat_hbm.at[idx_v.at[pl.ds(j * 128, 128)]],
                         col_v.at[pl.ds(j * 128, 128)], sem)
        for j in range(ROWS_PER_W // 128)
    ]
    for c in copies:
        c.wait()

    vf = jnp.where(valid, jnp.float32(1.0), jnp.float32(0.0))

    def mul_body(c, carry):
        col_v[pl.ds(c * L, L)] = col_v[pl.ds(c * L, L)] * vf
        return carry

    lax.fori_loop(0, ROWS_PER_W // L, mul_body, jnp.int32(0))
    pltpu.sync_copy(col_v, out_hbm.at[pl.ds(base, ROWS_PER_W)])


@jax.jit
def _pwc_sc(times_pad, t16, vflat):
    mesh = plsc.VectorSubcoreMesh(core_axis_name="c", subcore_axis_name="s")
    f = functools.partial(
        pl.kernel,
        mesh=mesh,
        out_type=jax.ShapeDtypeStruct((BATCH_K,), jnp.float32),
        scratch_types=[
            pltpu.VMEM((N_TIMES_PAD,), jnp.float32),
            pltpu.VMEM((L,), jnp.float32),
            pltpu.VMEM((ROWS_PER_W,), jnp.int32),
            pltpu.VMEM((ROWS_PER_W,), jnp.float32),
            pltpu.SemaphoreType.DMA,
        ],
        compiler_params=pltpu.CompilerParams(needs_layout_passes=False),
    )(_pwc_body)
    return f(times_pad, t16, vflat)


def kernel(times, values, t):
    times_pad = jnp.concatenate(
        [times, jnp.full((N_TIMES_PAD - N_BINS_K - 1,), jnp.inf, jnp.float32)]
    )
    t16 = jnp.broadcast_to(t, (L,)).astype(jnp.float32)
    vflat = values.reshape(-1)
    return _pwc_sc(times_pad, t16, vflat)


# speedup vs baseline: 11.0330x; 7.9499x over previous
import functools

import jax
import jax.numpy as jnp
from jax import lax
from jax.experimental import pallas as pl
from jax.experimental.pallas import tpu as pltpu
from jax.experimental.pallas import tpu_sc as plsc

N_BINS_K = 4096
BATCH_K = 16384
L = 16
NW = 32
ROWS_PER_W = BATCH_K // NW
N_TIMES_PAD = 4112
N_CHUNKS = N_TIMES_PAD // L


def _pwc_body(times_hbm, t_hbm, values_hbm, out_hbm,
              times_v, t_v, tile_v, col_v, sem):
    wid = lax.axis_index("s") * 2 + lax.axis_index("c")
    base = wid * ROWS_PER_W

    pltpu.sync_copy(times_hbm, times_v)
    pltpu.sync_copy(t_hbm, t_v)
    tval = t_v[...]

    def cnt_body(i, acc):
        chunk = times_v[pl.ds(i * L, L)]
        return acc + jnp.where(chunk <= tval, jnp.int32(1), jnp.int32(0))

    acc = lax.fori_loop(0, N_CHUNKS, cnt_body, jnp.zeros((L,), jnp.int32))
    count = jnp.sum(acc)
    valid = jnp.logical_and(count >= 1, count <= N_BINS_K)
    col = jnp.clip(count - 1, 0, N_BINS_K - 1)

    col_tile = pl.multiple_of((col // 128) * 128, 128)
    pltpu.async_copy(
        values_hbm.at[pl.ds(pl.multiple_of(base, 8), ROWS_PER_W),
                      pl.ds(col_tile, 128)],
        tile_v, sem).wait()

    vf = jnp.where(valid, jnp.float32(1.0), jnp.float32(0.0))
    colrem = jnp.full((L,), col % 128, jnp.int32)
    lanes = lax.iota(jnp.int32, L)

    def sel_body(c, carry):
        rows = c * L + lanes
        col_v[pl.ds(c * L, L)] = (
            plsc.load_gather(tile_v, [rows, colrem]) * vf)
        return carry

    lax.fori_loop(0, ROWS_PER_W // L, sel_body, jnp.int32(0))
    pltpu.sync_copy(col_v, out_hbm.at[pl.ds(base, ROWS_PER_W)])


@jax.jit
def _pwc_sc(times_pad, t16, values):
    mesh = plsc.VectorSubcoreMesh(core_axis_name="c", subcore_axis_name="s")
    f = functools.partial(
        pl.kernel,
        mesh=mesh,
        out_type=jax.ShapeDtypeStruct((BATCH_K,), jnp.float32),
        scratch_types=[
            pltpu.VMEM((N_TIMES_PAD,), jnp.float32),
            pltpu.VMEM((L,), jnp.float32),
            pltpu.VMEM((ROWS_PER_W, 128), jnp.float32),
            pltpu.VMEM((ROWS_PER_W,), jnp.float32),
            pltpu.SemaphoreType.DMA,
        ],
        compiler_params=pltpu.CompilerParams(needs_layout_passes=False),
    )(_pwc_body)
    return f(times_pad, t16, values)


def kernel(times, values, t):
    times_pad = jnp.concatenate(
        [times, jnp.full((N_TIMES_PAD - N_BINS_K - 1,), jnp.inf, jnp.float32)]
    )
    t16 = jnp.broadcast_to(t, (L,)).astype(jnp.float32)
    return _pwc_sc(times_pad, t16, values)


# speedup vs baseline: 11.5894x; 1.0504x over previous
import functools

import jax
import jax.numpy as jnp
from jax import lax
from jax.experimental import pallas as pl
from jax.experimental.pallas import tpu as pltpu
from jax.experimental.pallas import tpu_sc as plsc

N_BINS_K = 4096
BATCH_K = 16384
L = 16
NW = 32
ROWS_PER_W = BATCH_K // NW
N_TIMES = N_BINS_K + 1


def _pwc_body(times_hbm, t_hbm, values_hbm, out_hbm,
              times_v, t_v, tile_v, col_v, sem_a, sem_b):
    wid = lax.axis_index("s") * 2 + lax.axis_index("c")
    base = wid * ROWS_PER_W

    cp_times = pltpu.async_copy(times_hbm, times_v, sem_a)
    cp_t = pltpu.async_copy(t_hbm, t_v, sem_b)
    cp_times.wait()
    cp_t.wait()

    lanes = lax.iota(jnp.int32, L)
    zeros = jnp.zeros((L,), jnp.int32)
    tval = plsc.load_gather(t_v, [zeros])

    def probe(idx):
        smp = plsc.load_gather(times_v, [jnp.minimum(idx, N_BINS_K)])
        ok = jnp.logical_and(smp <= tval, idx <= N_BINS_K)
        return jnp.sum(jnp.where(ok, jnp.int32(1), jnp.int32(0)))

    n1 = probe(lanes * 256 + 255)
    b1 = n1 * 256
    n2 = probe(b1 + lanes * 16 + 15)
    b2 = b1 + n2 * 16
    n3 = probe(b2 + lanes)
    count = b2 + n3

    valid = jnp.logical_and(count >= 1, count <= N_BINS_K)
    col = jnp.clip(count - 1, 0, N_BINS_K - 1)

    col_tile = pl.multiple_of((col // 128) * 128, 128)
    pltpu.async_copy(
        values_hbm.at[pl.ds(pl.multiple_of(base, 8), ROWS_PER_W),
                      pl.ds(col_tile, 128)],
        tile_v, sem_a).wait()

    vf = jnp.where(valid, jnp.float32(1.0), jnp.float32(0.0))
    colrem = jnp.full((L,), col % 128, jnp.int32)

    def sel_body(c, carry):
        rows = c * L + lanes
        col_v[pl.ds(c * L, L)] = (
            plsc.load_gather(tile_v, [rows, colrem]) * vf)
        return carry

    lax.fori_loop(0, ROWS_PER_W // L, sel_body, jnp.int32(0))
    pltpu.sync_copy(col_v, out_hbm.at[pl.ds(base, ROWS_PER_W)])


@jax.jit
def _pwc_sc(times, t1, values):
    mesh = plsc.VectorSubcoreMesh(core_axis_name="c", subcore_axis_name="s")
    f = functools.partial(
        pl.kernel,
        mesh=mesh,
        out_type=jax.ShapeDtypeStruct((BATCH_K,), jnp.float32),
        scratch_types=[
            pltpu.VMEM((N_TIMES,), jnp.float32),
            pltpu.VMEM((1,), jnp.float32),
            pltpu.VMEM((ROWS_PER_W, 128), jnp.float32),
            pltpu.VMEM((ROWS_PER_W,), jnp.float32),
            pltpu.SemaphoreType.DMA,
            pltpu.SemaphoreType.DMA,
        ],
        compiler_params=pltpu.CompilerParams(needs_layout_passes=False),
    )(_pwc_body)
    return f(times, t1, values)


def kernel(times, values, t):
    return _pwc_sc(times, jnp.reshape(t, (1,)), values)
